# Initial kernel scaffold; baseline (speedup 1.0000x reference)
import jax
import jax.numpy as jnp
from jax.experimental import pallas as pl


def kernel(pred, target):
    raise NotImplementedError("write your pallas kernel here")


# speedup vs baseline: 15.1985x; 15.1985x over previous
import jax
import jax.numpy as jnp
from jax import lax
from jax.experimental import pallas as pl
from jax.experimental.pallas import tpu as pltpu

_B = 4096
_C = 1000
_CPAD = 1024
_BLK = 128


def _body(pred_ref, targ_ref, out_ref):
    p = pred_ref[...]
    t = targ_ref[...]
    loss = jnp.maximum(p, 0.0) - p * t + jnp.log1p(jnp.exp(-jnp.abs(p)))

    pos = jnp.sum(t, axis=0, keepdims=True)
    pos_gt = (pos * 2.0 >= float(_B)).astype(jnp.float32)
    majority = t == pos_gt
    n_maj = jnp.sum(majority.astype(jnp.int32), axis=0, keepdims=True)
    n_min = _B - n_maj

    minority_sum = jnp.sum(jnp.where(majority, 0.0, loss))

    bits = jnp.where(majority, lax.bitcast_convert_type(loss, jnp.int32),
                     jnp.int32(-1))

    def step(i, T):
        cand = T | (jnp.int32(1) << (30 - i))
        cnt = jnp.sum((bits >= cand).astype(jnp.int32), axis=0, keepdims=True)
        return jnp.where(cnt >= n_min, cand, T)

    T = lax.fori_loop(0, 31, step, jnp.zeros((1, _BLK), jnp.int32))

    gt = bits > T
    cnt_gt = jnp.sum(gt.astype(jnp.int32), axis=0, keepdims=True)
    sum_gt = jnp.sum(jnp.where(gt, loss, 0.0))
    tie = lax.bitcast_convert_type(T, jnp.float32)
    extra = jnp.where(n_min > 0, (n_min - cnt_gt).astype(jnp.float32) * tie, 0.0)

    @pl.when(pl.program_id(0) == 0)
    def _():
        out_ref[...] = jnp.zeros((1, 1), jnp.float32)

    total = minority_sum + sum_gt + jnp.sum(extra)
    out_ref[...] += jnp.reshape(total, (1, 1))


def kernel(pred, target):
    pad = _CPAD - _C
    predp = jnp.pad(pred, ((0, 0), (0, pad)))
    targp = jnp.pad(target, ((0, 0), (0, pad)))
    total = pl.pallas_call(
        _body,
        grid=(_CPAD // _BLK,),
        in_specs=[
            pl.BlockSpec((_B, _BLK), lambda j: (0, j)),
            pl.BlockSpec((_B, _BLK), lambda j: (0, j)),
        ],
        out_specs=pl.BlockSpec((1, 1), lambda j: (0, 0)),
        out_shape=jax.ShapeDtypeStruct((1, 1), jnp.float32),
        compiler_params=pltpu.CompilerParams(
            dimension_semantics=("arbitrary",),
        ),
    )(predp, targp)
    return total[0, 0] / jnp.float32(_B * _C)
